# baseline (speedup 1.0000x reference)
import jax
import jax.numpy as jnp
from jax.experimental import pallas as pl
from jax.experimental.pallas import tpu as pltpu

D_MODEL = 4096
N_INPUT = 8
N_PROCESS = 32
PROCESS_K = 3
BLOCK_T = 128
N_STREAMS = 8
LANES = 64


def _epilogue(s, idx_ref, wgt_ref, base):
    inp = s[:, 0:N_INPUT]
    proc = s[:, N_PROCESS:LANES]

    m = jnp.max(inp, axis=1, keepdims=True)
    e = jnp.exp(inp - m)
    wgt_ref[base : base + BLOCK_T, :] = e / jnp.sum(e, axis=1, keepdims=True)

    fiota = jax.lax.broadcasted_iota(jnp.int32, proc.shape, 1).astype(jnp.float32)
    cols = []
    for k in range(PROCESS_K):
        mx = jnp.max(proc, axis=1, keepdims=True)
        cand = jnp.where(proc == mx, fiota, 64.0)
        sel = jnp.min(cand, axis=1, keepdims=True)
        cols.append(sel)
        if k + 1 < PROCESS_K:
            proc = jnp.where(fiota == sel, -jnp.inf, proc)
    idx_ref[base : base + BLOCK_T, :] = jnp.concatenate(cols, axis=1).astype(
        jnp.int32
    )


def _router_kernel(*refs):
    x_refs = refs[:N_STREAMS]
    win_ref, wproc_ref = refs[N_STREAMS], refs[N_STREAMS + 1]
    idx_ref, wgt_ref = refs[N_STREAMS + 2], refs[N_STREAMS + 3]
    wt_ref = refs[N_STREAMS + 4]

    @pl.when(pl.program_id(0) == 0)
    def _init():
        wt_ref[:, 0:N_INPUT] = win_ref[...].T
        wt_ref[:, N_PROCESS:LANES] = wproc_ref[...].T

    w = wt_ref[...]
    dn = (((1,), (0,)), ((), ()))
    for j, x_ref in enumerate(x_refs):
        s = jax.lax.dot_general(x_ref[...], w, dn, preferred_element_type=jnp.float32)
        _epilogue(s, idx_ref, wgt_ref, j * BLOCK_T)


@jax.jit
def kernel(x, W_in, W_proc):
    B, S, D = x.shape
    T = B * S
    xf = x.reshape(T, D)
    idx, wgt = pl.pallas_call(
        _router_kernel,
        grid=(T // (N_STREAMS * BLOCK_T),),
        compiler_params=pltpu.CompilerParams(
            dimension_semantics=("arbitrary",),
        ),
        in_specs=[
            pl.BlockSpec((BLOCK_T, D), lambda i, j=j: (N_STREAMS * i + j, 0))
            for j in range(N_STREAMS)
        ]
        + [
            pl.BlockSpec((N_INPUT, D), lambda i: (0, 0)),
            pl.BlockSpec((N_PROCESS, D), lambda i: (0, 0)),
        ],
        out_specs=[
            pl.BlockSpec((N_STREAMS * BLOCK_T, PROCESS_K), lambda i: (i, 0)),
            pl.BlockSpec((N_STREAMS * BLOCK_T, N_INPUT), lambda i: (i, 0)),
        ],
        out_shape=[
            jax.ShapeDtypeStruct((T, PROCESS_K), jnp.int32),
            jax.ShapeDtypeStruct((T, N_INPUT), jnp.float32),
        ],
        scratch_shapes=[pltpu.VMEM((D_MODEL, LANES), jnp.float32)],
    )(*([xf] * N_STREAMS), W_in, W_proc)
    return idx.reshape(B, S, PROCESS_K), wgt.reshape(B, S, N_INPUT)


# speedup vs baseline: 1.0963x; 1.0963x over previous
import jax
import jax.numpy as jnp
from jax.experimental import pallas as pl
from jax.experimental.pallas import tpu as pltpu

D_MODEL = 4096
N_INPUT = 8
N_PROCESS = 32
PROCESS_K = 3
BLOCK_T = 256
N_STREAMS = 4
LANES = 64


def _epilogue(s, idx_ref, wgt_ref, base):
    inp = s[:, 0:N_INPUT]
    proc = s[:, N_PROCESS:LANES]

    m = jnp.max(inp, axis=1, keepdims=True)
    e = jnp.exp(inp - m)
    wgt_ref[base : base + BLOCK_T, :] = e / jnp.sum(e, axis=1, keepdims=True)

    fiota = jax.lax.broadcasted_iota(jnp.int32, proc.shape, 1).astype(jnp.float32)
    cols = []
    for k in range(PROCESS_K):
        mx = jnp.max(proc, axis=1, keepdims=True)
        cand = jnp.where(proc == mx, fiota, 64.0)
        sel = jnp.min(cand, axis=1, keepdims=True)
        cols.append(sel)
        if k + 1 < PROCESS_K:
            proc = jnp.where(fiota == sel, -jnp.inf, proc)
    idx_ref[base : base + BLOCK_T, :] = jnp.concatenate(cols, axis=1).astype(
        jnp.int32
    )


def _router_kernel(*refs):
    x_refs = refs[:N_STREAMS]
    win_ref, wproc_ref = refs[N_STREAMS], refs[N_STREAMS + 1]
    idx_ref, wgt_ref = refs[N_STREAMS + 2], refs[N_STREAMS + 3]
    wt_ref = refs[N_STREAMS + 4]

    @pl.when(pl.program_id(0) == 0)
    def _init():
        wt_ref[:, 0:N_INPUT] = win_ref[...].T
        wt_ref[:, N_PROCESS:LANES] = wproc_ref[...].T

    w = wt_ref[...]
    dn = (((1,), (0,)), ((), ()))
    for j, x_ref in enumerate(x_refs):
        s = jax.lax.dot_general(x_ref[...], w, dn, preferred_element_type=jnp.float32)
        _epilogue(s, idx_ref, wgt_ref, j * BLOCK_T)


@jax.jit
def kernel(x, W_in, W_proc):
    B, S, D = x.shape
    T = B * S
    xf = x.reshape(T, D)
    idx, wgt = pl.pallas_call(
        _router_kernel,
        grid=(T // (N_STREAMS * BLOCK_T),),
        compiler_params=pltpu.CompilerParams(
            dimension_semantics=("arbitrary",),
        ),
        in_specs=[
            pl.BlockSpec((BLOCK_T, D), lambda i, j=j: (N_STREAMS * i + j, 0))
            for j in range(N_STREAMS)
        ]
        + [
            pl.BlockSpec((N_INPUT, D), lambda i: (0, 0)),
            pl.BlockSpec((N_PROCESS, D), lambda i: (0, 0)),
        ],
        out_specs=[
            pl.BlockSpec((N_STREAMS * BLOCK_T, PROCESS_K), lambda i: (i, 0)),
            pl.BlockSpec((N_STREAMS * BLOCK_T, N_INPUT), lambda i: (i, 0)),
        ],
        out_shape=[
            jax.ShapeDtypeStruct((T, PROCESS_K), jnp.int32),
            jax.ShapeDtypeStruct((T, N_INPUT), jnp.float32),
        ],
        scratch_shapes=[pltpu.VMEM((D_MODEL, LANES), jnp.float32)],
    )(*([xf] * N_STREAMS), W_in, W_proc)
    return idx.reshape(B, S, PROCESS_K), wgt.reshape(B, S, N_INPUT)
